# baseline (speedup 1.0000x reference)
import functools

import jax
import jax.numpy as jnp
from jax import lax
from jax.experimental import pallas as pl
from jax.experimental.pallas import tpu as pltpu
from jax.experimental.pallas import tpu_sc as plsc

N = 10000
E = 320000
D = 128
DE = 16
H = 4
C = 32
B = 64

NC = 2
NS = 16
NW = NC * NS
EW = E // NW
CH = 96
NCH = EW // CH
TAIL = EW - NCH * CH
NR = 144
ZTILE = 632
ZLAST = N - 15 * ZTILE


def _sc_edge_pass(table, adst, src, dst, e4):
    mesh = plsc.VectorSubcoreMesh(core_axis_name="c", subcore_axis_name="s")

    @functools.partial(
        pl.kernel,
        out_type=jax.ShapeDtypeStruct((NC, N, NR), jnp.float32),
        mesh=mesh,
        scratch_types=[
            pltpu.VMEM_SHARED((N, NR), jnp.float32),
            pltpu.VMEM((CH, NR), jnp.float32),
            pltpu.VMEM((CH, NR), jnp.float32),
            pltpu.VMEM((CH,), jnp.int32),
            pltpu.VMEM((CH,), jnp.int32),
            pltpu.VMEM((CH,), jnp.int32),
            pltpu.VMEM((CH,), jnp.int32),
            pltpu.VMEM((CH, 16), jnp.float32),
            pltpu.VMEM((CH, 16), jnp.float32),
            pltpu.VMEM((CH, 4), jnp.float32),
            pltpu.VMEM((CH, 4), jnp.float32),
            pltpu.VMEM((TAIL,), jnp.int32),
            pltpu.VMEM((TAIL,), jnp.int32),
            pltpu.SemaphoreType.DMA,
            pltpu.SemaphoreType.DMA,
            pltpu.SemaphoreType.DMA,
            pltpu.SemaphoreType.DMA,
        ],
        compiler_params=pltpu.CompilerParams(use_tc_tiling_on_sc=False,
                                             needs_layout_passes=False),
    )
    def body(table_h, adst_h, src_h, dst_h, e4_h, out_h,
             acc, rows0, rows1, src0, src1, dst0, dst1, adr0, adr1,
             e40, e41, srct, dstt, semr0, semr1, sema0, sema1):
        cid = lax.axis_index("c")
        sid = lax.axis_index("s")
        wid = sid * NC + cid
        lane = lax.broadcasted_iota(jnp.int32, (16,), 0)
        el = lane >> 2
        hh = lane & 3
        zv = jnp.zeros((16,), jnp.float32)
        rows = [rows0, rows1]
        srci = [src0, src1]
        dsti = [dst0, dst1]
        adr = [adr0, adr1]
        e4v = [e40, e41]
        semr = [semr0, semr1]
        sema = [sema0, sema1]

        def zrow(r, _):
            for cgrp in range(NR // 16):
                rows0[r, pl.ds(cgrp * 16, 16)] = zv
            return 0

        lax.fori_loop(0, CH, zrow, 0)
        tb = sid * ZTILE
        for k in range(5):
            off = pl.multiple_of(tb + k * CH, 8)
            pltpu.sync_copy(rows0, acc.at[pl.ds(off, CH)])
        off5 = pl.multiple_of(tb + 5 * CH, 8)
        off6 = pl.multiple_of(tb + 6 * CH, 8)

        @pl.when(sid < NS - 1)
        def _():
            pltpu.sync_copy(rows0, acc.at[pl.ds(off5, CH)])
            pltpu.sync_copy(rows0.at[pl.ds(0, ZTILE - 6 * CH)],
                            acc.at[pl.ds(off6, ZTILE - 6 * CH)])

        @pl.when(sid == NS - 1)
        def _():
            pltpu.sync_copy(rows0.at[pl.ds(0, ZLAST - 5 * CH)],
                            acc.at[pl.ds(off5, ZLAST - 5 * CH)])

        plsc.subcore_barrier()

        def load_idx(g, b):
            base = wid * EW + g * CH
            pltpu.sync_copy(src_h.at[pl.ds(base, CH)], srci[b])
            pltpu.sync_copy(dst_h.at[pl.ds(base, CH)], dsti[b])
            pltpu.sync_copy(e4_h.at[pl.ds(base, CH)], e4v[b])

        def start_gather(b):
            pltpu.async_copy(table_h.at[srci[b]], rows[b], semr[b])
            pltpu.async_copy(adst_h.at[dsti[b]], adr[b], sema[b])

        def wait_gather(b):
            pltpu.make_async_copy(table_h.at[srci[b]], rows[b], semr[b]).wait()
            pltpu.make_async_copy(adst_h.at[dsti[b]], adr[b], sema[b]).wait()

        def compute(nch4, rbuf, abuf, ebuf):
            @plsc.parallel_loop(0, nch4, unroll=4)
            def pstep(j):
                rowv = j * 4 + el
                asrc = plsc.load_gather(rbuf, [rowv, 128 + hh])
                ad = plsc.load_gather(abuf, [rowv, hh])
                ev = plsc.load_gather(ebuf, [rowv, hh])
                al = asrc + ad + ev
                al = jnp.maximum(al, al * 0.2)
                pv = jnp.exp(al)
                plsc.store_scatter(rbuf, [rowv, 128 + hh], pv)
                plsc.store_scatter(rbuf, [rowv, 132 + hh], ev)
                for t in range(4):
                    e = j * 4 + t
                    for h in range(H):
                        ps = pv[t * 4 + h]
                        for q in range(2):
                            off = h * 32 + q * 16
                            rbuf[e, pl.ds(off, 16)] = (
                                rbuf[e, pl.ds(off, 16)] * ps)

        load_idx(0, 0)
        start_gather(0)

        def pair(i, _):
            for b in range(2):
                g = i * 2 + b
                nb = 1 - b

                @pl.when(g < NCH - 1)
                def _():
                    load_idx(g + 1, nb)
                    start_gather(nb)

                wait_gather(b)
                compute(CH // 4, rows[b], adr[b], e4v[b])
                pltpu.sync_copy(rows[b], acc.at[dsti[b]], add=True)
            return 0

        lax.fori_loop(0, NCH // 2, pair, 0)

        tbase = wid * EW + NCH * CH
        pltpu.sync_copy(src_h.at[pl.ds(tbase, TAIL)], srct)
        pltpu.sync_copy(dst_h.at[pl.ds(tbase, TAIL)], dstt)
        pltpu.sync_copy(e4_h.at[pl.ds(tbase, TAIL)], e40.at[pl.ds(0, TAIL)])
        cp1 = pltpu.async_copy(table_h.at[srct], rows0.at[pl.ds(0, TAIL)],
                               semr0)
        cp2 = pltpu.async_copy(adst_h.at[dstt], adr0.at[pl.ds(0, TAIL)],
                               sema0)
        cp1.wait()
        cp2.wait()
        compute(TAIL // 4, rows0, adr0, e40)
        pltpu.sync_copy(rows0.at[pl.ds(0, TAIL)], acc.at[dstt], add=True)

        plsc.subcore_barrier()
        doff = pl.multiple_of(sid * ZTILE, 8)

        @pl.when(sid < NS - 1)
        def _():
            pltpu.sync_copy(acc.at[pl.ds(doff, ZTILE)],
                            out_h.at[cid, pl.ds(doff, ZTILE)])

        @pl.when(sid == NS - 1)
        def _():
            pltpu.sync_copy(acc.at[pl.ds(doff, ZLAST)],
                            out_h.at[cid, pl.ds(doff, ZLAST)])

    return body(table, adst, src, dst, e4)


BN = 400
BE = 4000
HIGH = jax.lax.Precision.HIGHEST


def _dot(a, b):
    return jnp.dot(a, b, precision=HIGH, preferred_element_type=jnp.float32)


def _dotd(a, b):
    return jnp.dot(a, b, preferred_element_type=jnp.float32)


def _ln(x, g, b, eps=1e-5):
    mu = jnp.mean(x, axis=-1, keepdims=True)
    var = jnp.mean((x - mu) ** 2, axis=-1, keepdims=True)
    return (x - mu) / jnp.sqrt(var + eps) * g + b


def _elu(x):
    return jnp.where(x > 0, x, jnp.exp(x) - 1.0)


def _row_spec(width):
    return pl.BlockSpec((BN, width), lambda i: (i, 0))


def _full_spec(shape):
    nd = len(shape)
    return pl.BlockSpec(shape, lambda i: (0,) * nd)


def _tables_from_xw(xw, matt):
    s = _dot(xw, matt)
    zero4 = jnp.zeros((xw.shape[0], 4), jnp.float32)
    one1 = jnp.ones((xw.shape[0], 1), jnp.float32)
    zero7 = jnp.zeros((xw.shape[0], 7), jnp.float32)
    zero12 = jnp.zeros((xw.shape[0], 12), jnp.float32)
    table = jnp.concatenate([xw, s[:, 0:4], zero4, one1, zero7], axis=1)
    adst16 = jnp.concatenate([s[:, 4:8], zero12], axis=1)
    return table, adst16


def _pre_kernel(x_ref, win_ref, bin_ref, w_ref, matt_ref,
                table_ref, adst_ref):
    x = x_ref[...]
    h0 = jax.nn.relu(_dotd(x, win_ref[...]) + bin_ref[...])
    xw = _dotd(jnp.concatenate([x, h0], axis=1), w_ref[...])
    table, adst16 = _tables_from_xw(xw, matt_ref[...])
    table_ref[...] = table
    adst_ref[...] = adst16


def _e4_kernel(ea_ref, we1_ref, att1_ref, we2_ref, att2_ref, kron_ref,
               o_ref):
    ea = ea_ref[...]
    e41 = _dot(_dotd(ea, we1_ref[...]) * att1_ref[...], kron_ref[...])
    e42 = _dot(_dotd(ea, we2_ref[...]) * att2_ref[...], kron_ref[...])
    o_ref[...] = jnp.concatenate([e41, e42], axis=1)


def _combine(parts_ref, table_ref, adst_ref, erep_ref, bias_ref, lng_ref,
             lnb_ref):
    s = parts_ref[0] + parts_ref[1]
    xw = table_ref[:, 0:128]
    a_src = table_ref[:, 128:132]
    a_dst = adst_ref[:, 0:4]
    num = s[:, 0:128]
    den = s[:, 128:132]
    se4 = s[:, 132:136]
    deg = s[:, 136:137]
    loop_a = se4 / jnp.maximum(deg, 1.0)
    al = a_src + a_dst + loop_a
    p_self = jnp.exp(jnp.maximum(al, 0.2 * al))
    den_t = _dot(den + p_self, erep_ref[...]) + 1e-16
    num_t = num + xw * _dot(p_self, erep_ref[...])
    out = num_t / den_t + bias_ref[...]
    return _elu(_ln(out, lng_ref[...], lnb_ref[...]))


def _mid_kernel(parts_ref, table_ref, adst_ref, x_ref, erep_ref, bias_ref,
                lng_ref, lnb_ref, w_ref, matt_ref,
                table2_ref, adst2_ref):
    h1 = _combine(parts_ref, table_ref, adst_ref, erep_ref, bias_ref,
                  lng_ref, lnb_ref)
    xw2 = _dotd(jnp.concatenate([x_ref[...], h1], axis=1), w_ref[...])
    table2, adst16 = _tables_from_xw(xw2, matt_ref[...])
    table2_ref[...] = table2
    adst2_ref[...] = adst16


def _post_kernel(parts_ref, table_ref, adst_ref, x_ref, batch_ref, erep_ref,
                 bias_ref, lng_ref, lnb_ref, wg_ref, bg_ref, lngg_ref,
                 lngb_ref, wo_ref, bo_ref, lnlg_ref, lnlb_ref,
                 wn1_ref, bn1_ref, wn2_ref, bn2_ref, wn3_ref,
                 bn3_ref, no_ref, pool_ref):
    i = pl.program_id(0)
    h2 = _combine(parts_ref, table_ref, adst_ref, erep_ref, bias_ref,
                  lng_ref, lnb_ref)
    ge = _ln(jax.nn.relu(_dotd(h2, wg_ref[...]) + bg_ref[...]),
             lngg_ref[...], lngb_ref[...])
    bid = batch_ref[0, 0, :]
    mask = (jax.lax.broadcasted_iota(jnp.int32, (B, BN), 0)
            == bid[None, :]).astype(jnp.float32)
    geaug = jnp.concatenate(
        [ge, jnp.ones((BN, 1), jnp.float32),
         jnp.zeros((BN, 127), jnp.float32)], axis=1)
    part = _dot(mask, geaug)

    @pl.when(i == 0)
    def _():
        pool_ref[...] = jnp.zeros_like(pool_ref)

    pool_ref[...] += part

    ho = _ln(jax.nn.relu(_dotd(h2, wo_ref[...]) + bo_ref[...]),
             lnlg_ref[...], lnlb_ref[...])
    z = jax.nn.relu(_dotd(jnp.concatenate([x_ref[...], ho], axis=1),
                          wn1_ref[...]) + bn1_ref[...])
    z = jax.nn.relu(_dotd(z, wn2_ref[...]) + bn2_ref[...])
    no_ref[...] = _dotd(z, wn3_ref[...]) + bn3_ref[...]


def _graph_kernel(pool_ref, wg1_ref, bg1_ref, wg2_ref, bg2_ref, wg3_ref,
                  bg3_ref, go_ref):
    pooled = pool_ref[...]
    ge = pooled[:, 0:128] / jnp.maximum(pooled[:, 128:129], 1.0)
    z = jax.nn.relu(_dotd(ge, wg1_ref[...]) + bg1_ref[...])
    z = jax.nn.relu(_dotd(z, wg2_ref[...]) + bg2_ref[...])
    go_ref[...] = _dotd(z, wg3_ref[...]) + bg3_ref[...]


def _blockdiag_att(att):
    maskm = jnp.kron(jnp.eye(H, dtype=jnp.float32),
                     jnp.ones((C, 1), jnp.float32))
    return att.reshape(H * C, 1) * maskm


def kernel(x, edge_attr, params, edge_index, batch):
    p = params
    src, dst = edge_index[0], edge_index[1]

    matts, a16s = [], []
    for l in range(2):
        g = p['gat'][l]
        matts.append(jnp.concatenate(
            [_blockdiag_att(g['att_src']), _blockdiag_att(g['att_dst'])],
            axis=1))
        a16s.append(g['att_e'].reshape(1, H * C))
    erep = jnp.kron(jnp.eye(H, dtype=jnp.float32),
                    jnp.ones((1, C), jnp.float32))
    kronm = jnp.kron(jnp.eye(H, dtype=jnp.float32),
                     jnp.ones((C, 1), jnp.float32))
    padc = lambda w, n: jnp.pad(w, ((0, 0), (0, n - w.shape[1])))
    padv = lambda v: jnp.pad(v, (0, 7)).reshape(1, 8)
    batch3d = batch.reshape(N // BN, 1, BN)

    grid_n = N // BN
    row128 = _row_spec(128)
    vec = lambda v, w: v.reshape(1, w)

    table1, adst1 = pl.pallas_call(
        _pre_kernel,
        grid=(grid_n,),
        in_specs=[row128, _full_spec((D, D)), _full_spec((1, D)),
                  _full_spec((2 * D, D)), _full_spec((D, 8))],
        out_specs=[_row_spec(NR), _row_spec(16)],
        out_shape=[jax.ShapeDtypeStruct((N, NR), jnp.float32),
                   jax.ShapeDtypeStruct((N, 16), jnp.float32)],
    )(x, p['W_in'], vec(p['b_in'], D), p['gat'][0]['W'], matts[0])

    e4pack = pl.pallas_call(
        _e4_kernel,
        grid=(E // BE,),
        in_specs=[pl.BlockSpec((BE, DE), lambda i: (i, 0)),
                  _full_spec((DE, D)), _full_spec((1, D)),
                  _full_spec((DE, D)), _full_spec((1, D)),
                  _full_spec((D, H))],
        out_specs=pl.BlockSpec((BE, 2 * H), lambda i: (i, 0)),
        out_shape=jax.ShapeDtypeStruct((E, 2 * H), jnp.float32),
    )(edge_attr, p['gat'][0]['W_e'], a16s[0], p['gat'][1]['W_e'], a16s[1],
      kronm)
    e4s = [e4pack[:, 0:4], e4pack[:, 4:8]]

    parts1 = _sc_edge_pass(table1, adst1, src, dst, e4s[0])
    g1 = p['gat'][0]
    table2, adst2 = pl.pallas_call(
        _mid_kernel,
        grid=(grid_n,),
        in_specs=[pl.BlockSpec((NC, BN, NR), lambda i: (0, i, 0)),
                  _row_spec(NR), _row_spec(16), row128,
                  _full_spec((H, D)), _full_spec((1, D)),
                  _full_spec((1, D)), _full_spec((1, D)),
                  _full_spec((2 * D, D)), _full_spec((D, 8))],
        out_specs=[_row_spec(NR), _row_spec(16)],
        out_shape=[jax.ShapeDtypeStruct((N, NR), jnp.float32),
                   jax.ShapeDtypeStruct((N, 16), jnp.float32)],
    )(parts1, table1, adst1, x, erep, vec(g1['bias'], D),
      vec(g1['ln_g'], D), vec(g1['ln_b'], D), p['gat'][1]['W'], matts[1])

    parts2 = _sc_edge_pass(table2, adst2, src, dst, e4s[1])
    g2 = p['gat'][1]
    no_full, pooled = pl.pallas_call(
        _post_kernel,
        grid=(grid_n,),
        in_specs=[pl.BlockSpec((NC, BN, NR), lambda i: (0, i, 0)),
                  _row_spec(NR), _row_spec(16), row128,
                  pl.BlockSpec((1, 1, BN), lambda i: (i, 0, 0)),
                  _full_spec((H, D)), _full_spec((1, D)),
                  _full_spec((1, D)), _full_spec((1, D)),
                  _full_spec((D, D)), _full_spec((1, D)),
                  _full_spec((1, D)), _full_spec((1, D)),
                  _full_spec((D, D)), _full_spec((1, D)),
                  _full_spec((1, D)), _full_spec((1, D)),
                  _full_spec((2 * D, 32)),
                  _full_spec((1, 32)), _full_spec((32, 8)),
                  _full_spec((1, 8)), _full_spec((8, 8)),
                  _full_spec((1, 8))],
        out_specs=[pl.BlockSpec((BN, 8), lambda i: (i, 0)),
                   pl.BlockSpec((B, 256), lambda i: (0, 0))],
        out_shape=[jax.ShapeDtypeStruct((N, 8), jnp.float32),
                   jax.ShapeDtypeStruct((B, 256), jnp.float32)],
    )(parts2, table2, adst2, x, batch3d, erep, vec(g2['bias'], D),
      vec(g2['ln_g'], D), vec(g2['ln_b'], D),
      p['Wg'], vec(p['bg'], D), vec(p['lng_g'], D), vec(p['lng_b'], D),
      p['Wo'], vec(p['bo'], D), vec(p['lnl_g'], D), vec(p['lnl_b'], D),
      p['Wn1'], vec(p['bn1'], 32), p['Wn2'], vec(p['bn2'], 8),
      padc(p['Wn3'], 8), padv(p['bn3']))

    go_full = pl.pallas_call(
        _graph_kernel,
        grid=(1,),
        in_specs=[_full_spec((B, 256)), _full_spec((D, 32)),
                  _full_spec((1, 32)), _full_spec((32, 8)),
                  _full_spec((1, 8)), _full_spec((8, 8)),
                  _full_spec((1, 8))],
        out_specs=pl.BlockSpec((B, 8), lambda i: (0, 0)),
        out_shape=jax.ShapeDtypeStruct((B, 8), jnp.float32),
    )(pooled, p['Wg1'], vec(p['bg1'], 32), p['Wg2'], vec(p['bg2'], 8),
      padc(p['Wg3'], 8), padv(p['bg3']))

    return jnp.concatenate([no_full[:, 0], go_full[:, 0]], axis=-1)


# speedup vs baseline: 1.0131x; 1.0131x over previous
import functools

import jax
import jax.numpy as jnp
from jax import lax
from jax.experimental import pallas as pl
from jax.experimental.pallas import tpu as pltpu
from jax.experimental.pallas import tpu_sc as plsc

N = 10000
E = 320000
D = 128
DE = 16
H = 4
C = 32
B = 64

NC = 2
NS = 16
NW = NC * NS
EW = E // NW
CH = 96
NCH = EW // CH
TAIL = EW - NCH * CH
NR = 144
ZTILE = 632
ZLAST = N - 15 * ZTILE


def _sc_edge_pass(table, adst, src, dst, e4):
    mesh = plsc.VectorSubcoreMesh(core_axis_name="c", subcore_axis_name="s")

    @functools.partial(
        pl.kernel,
        out_type=jax.ShapeDtypeStruct((NC, N, NR), jnp.float32),
        mesh=mesh,
        scratch_types=[
            pltpu.VMEM_SHARED((N, NR), jnp.float32),
            pltpu.VMEM((CH, NR), jnp.float32),
            pltpu.VMEM((CH, NR), jnp.float32),
            pltpu.VMEM((CH,), jnp.int32),
            pltpu.VMEM((CH,), jnp.int32),
            pltpu.VMEM((CH,), jnp.int32),
            pltpu.VMEM((CH,), jnp.int32),
            pltpu.VMEM((CH, 16), jnp.float32),
            pltpu.VMEM((CH, 16), jnp.float32),
            pltpu.VMEM((CH, 4), jnp.float32),
            pltpu.VMEM((CH, 4), jnp.float32),
            pltpu.VMEM((TAIL,), jnp.int32),
            pltpu.VMEM((TAIL,), jnp.int32),
            pltpu.SemaphoreType.DMA,
            pltpu.SemaphoreType.DMA,
            pltpu.SemaphoreType.DMA,
            pltpu.SemaphoreType.DMA,
        ],
        compiler_params=pltpu.CompilerParams(use_tc_tiling_on_sc=False,
                                             needs_layout_passes=False),
    )
    def body(table_h, adst_h, src_h, dst_h, e4_h, out_h,
             acc, rows0, rows1, src0, src1, dst0, dst1, adr0, adr1,
             e40, e41, srct, dstt, semr0, semr1, sema0, sema1):
        cid = lax.axis_index("c")
        sid = lax.axis_index("s")
        wid = sid * NC + cid
        lane = lax.broadcasted_iota(jnp.int32, (16,), 0)
        el = lane >> 2
        hh = lane & 3
        zv = jnp.zeros((16,), jnp.float32)
        rows = [rows0, rows1]
        srci = [src0, src1]
        dsti = [dst0, dst1]
        adr = [adr0, adr1]
        e4v = [e40, e41]
        semr = [semr0, semr1]
        sema = [sema0, sema1]

        def zrow(r, _):
            for cgrp in range(NR // 16):
                rows0[r, pl.ds(cgrp * 16, 16)] = zv
            return 0

        lax.fori_loop(0, CH, zrow, 0)
        tb = sid * ZTILE
        for k in range(5):
            off = pl.multiple_of(tb + k * CH, 8)
            pltpu.sync_copy(rows0, acc.at[pl.ds(off, CH)])
        off5 = pl.multiple_of(tb + 5 * CH, 8)
        off6 = pl.multiple_of(tb + 6 * CH, 8)

        @pl.when(sid < NS - 1)
        def _():
            pltpu.sync_copy(rows0, acc.at[pl.ds(off5, CH)])
            pltpu.sync_copy(rows0.at[pl.ds(0, ZTILE - 6 * CH)],
                            acc.at[pl.ds(off6, ZTILE - 6 * CH)])

        @pl.when(sid == NS - 1)
        def _():
            pltpu.sync_copy(rows0.at[pl.ds(0, ZLAST - 5 * CH)],
                            acc.at[pl.ds(off5, ZLAST - 5 * CH)])

        plsc.subcore_barrier()

        def load_idx(g, b):
            base = wid * EW + g * CH
            pltpu.sync_copy(src_h.at[pl.ds(base, CH)], srci[b])
            pltpu.sync_copy(dst_h.at[pl.ds(base, CH)], dsti[b])
            pltpu.sync_copy(e4_h.at[pl.ds(base, CH)], e4v[b])

        def start_gather(b):
            pltpu.async_copy(table_h.at[srci[b]], rows[b], semr[b])
            pltpu.async_copy(adst_h.at[dsti[b]], adr[b], sema[b])

        def wait_gather(b):
            pltpu.make_async_copy(table_h.at[srci[b]], rows[b], semr[b]).wait()
            pltpu.make_async_copy(adst_h.at[dsti[b]], adr[b], sema[b]).wait()

        def compute(nch4, rbuf, abuf, ebuf):
            @plsc.parallel_loop(0, nch4, unroll=2)
            def pstep(j):
                rowv = j * 4 + el
                asrc = plsc.load_gather(rbuf, [rowv, 128 + hh])
                ad = plsc.load_gather(abuf, [rowv, hh])
                ev = plsc.load_gather(ebuf, [rowv, hh])
                al = asrc + ad + ev
                al = jnp.maximum(al, al * 0.2)
                pv = jnp.exp(al)
                plsc.store_scatter(rbuf, [rowv, 128 + hh], pv)
                plsc.store_scatter(rbuf, [rowv, 132 + hh], ev)
                for t in range(4):
                    e = j * 4 + t
                    for h in range(H):
                        ps = pv[t * 4 + h]
                        for q in range(2):
                            off = h * 32 + q * 16
                            rbuf[e, pl.ds(off, 16)] = (
                                rbuf[e, pl.ds(off, 16)] * ps)

        load_idx(0, 0)
        start_gather(0)

        def pair(i, _):
            for b in range(2):
                g = i * 2 + b
                nb = 1 - b

                @pl.when(g < NCH - 1)
                def _():
                    load_idx(g + 1, nb)
                    start_gather(nb)

                wait_gather(b)
                compute(CH // 4, rows[b], adr[b], e4v[b])
                pltpu.sync_copy(rows[b], acc.at[dsti[b]], add=True)
            return 0

        lax.fori_loop(0, NCH // 2, pair, 0)

        tbase = wid * EW + NCH * CH
        pltpu.sync_copy(src_h.at[pl.ds(tbase, TAIL)], srct)
        pltpu.sync_copy(dst_h.at[pl.ds(tbase, TAIL)], dstt)
        pltpu.sync_copy(e4_h.at[pl.ds(tbase, TAIL)], e40.at[pl.ds(0, TAIL)])
        cp1 = pltpu.async_copy(table_h.at[srct], rows0.at[pl.ds(0, TAIL)],
                               semr0)
        cp2 = pltpu.async_copy(adst_h.at[dstt], adr0.at[pl.ds(0, TAIL)],
                               sema0)
        cp1.wait()
        cp2.wait()
        compute(TAIL // 4, rows0, adr0, e40)
        pltpu.sync_copy(rows0.at[pl.ds(0, TAIL)], acc.at[dstt], add=True)

        plsc.subcore_barrier()
        doff = pl.multiple_of(sid * ZTILE, 8)

        @pl.when(sid < NS - 1)
        def _():
            pltpu.sync_copy(acc.at[pl.ds(doff, ZTILE)],
                            out_h.at[cid, pl.ds(doff, ZTILE)])

        @pl.when(sid == NS - 1)
        def _():
            pltpu.sync_copy(acc.at[pl.ds(doff, ZLAST)],
                            out_h.at[cid, pl.ds(doff, ZLAST)])

    return body(table, adst, src, dst, e4)


BN = 400
BE = 4000
HIGH = jax.lax.Precision.HIGHEST


def _dot(a, b):
    return jnp.dot(a, b, precision=HIGH, preferred_element_type=jnp.float32)


def _dotd(a, b):
    return jnp.dot(a, b, preferred_element_type=jnp.float32)


def _ln(x, g, b, eps=1e-5):
    mu = jnp.mean(x, axis=-1, keepdims=True)
    var = jnp.mean((x - mu) ** 2, axis=-1, keepdims=True)
    return (x - mu) / jnp.sqrt(var + eps) * g + b


def _elu(x):
    return jnp.where(x > 0, x, jnp.exp(x) - 1.0)


def _row_spec(width):
    return pl.BlockSpec((BN, width), lambda i: (i, 0))


def _full_spec(shape):
    nd = len(shape)
    return pl.BlockSpec(shape, lambda i: (0,) * nd)


def _tables_from_xw(xw, matt):
    s = _dot(xw, matt)
    zero4 = jnp.zeros((xw.shape[0], 4), jnp.float32)
    one1 = jnp.ones((xw.shape[0], 1), jnp.float32)
    zero7 = jnp.zeros((xw.shape[0], 7), jnp.float32)
    zero12 = jnp.zeros((xw.shape[0], 12), jnp.float32)
    table = jnp.concatenate([xw, s[:, 0:4], zero4, one1, zero7], axis=1)
    adst16 = jnp.concatenate([s[:, 4:8], zero12], axis=1)
    return table, adst16


def _pre_kernel(x_ref, win_ref, bin_ref, w_ref, matt_ref,
                table_ref, adst_ref):
    x = x_ref[...]
    h0 = jax.nn.relu(_dotd(x, win_ref[...]) + bin_ref[...])
    xw = _dotd(jnp.concatenate([x, h0], axis=1), w_ref[...])
    table, adst16 = _tables_from_xw(xw, matt_ref[...])
    table_ref[...] = table
    adst_ref[...] = adst16


def _e4_kernel(ea_ref, we1_ref, att1_ref, we2_ref, att2_ref, kron_ref,
               o_ref):
    ea = ea_ref[...]
    e41 = _dot(_dotd(ea, we1_ref[...]) * att1_ref[...], kron_ref[...])
    e42 = _dot(_dotd(ea, we2_ref[...]) * att2_ref[...], kron_ref[...])
    o_ref[...] = jnp.concatenate([e41, e42], axis=1)


def _combine(parts_ref, table_ref, adst_ref, erep_ref, bias_ref, lng_ref,
             lnb_ref):
    s = parts_ref[0] + parts_ref[1]
    xw = table_ref[:, 0:128]
    a_src = table_ref[:, 128:132]
    a_dst = adst_ref[:, 0:4]
    num = s[:, 0:128]
    den = s[:, 128:132]
    se4 = s[:, 132:136]
    deg = s[:, 136:137]
    loop_a = se4 / jnp.maximum(deg, 1.0)
    al = a_src + a_dst + loop_a
    p_self = jnp.exp(jnp.maximum(al, 0.2 * al))
    den_t = _dot(den + p_self, erep_ref[...]) + 1e-16
    num_t = num + xw * _dot(p_self, erep_ref[...])
    out = num_t / den_t + bias_ref[...]
    return _elu(_ln(out, lng_ref[...], lnb_ref[...]))


def _mid_kernel(parts_ref, table_ref, adst_ref, x_ref, erep_ref, bias_ref,
                lng_ref, lnb_ref, w_ref, matt_ref,
                table2_ref, adst2_ref):
    h1 = _combine(parts_ref, table_ref, adst_ref, erep_ref, bias_ref,
                  lng_ref, lnb_ref)
    xw2 = _dotd(jnp.concatenate([x_ref[...], h1], axis=1), w_ref[...])
    table2, adst16 = _tables_from_xw(xw2, matt_ref[...])
    table2_ref[...] = table2
    adst2_ref[...] = adst16


def _post_kernel(parts_ref, table_ref, adst_ref, x_ref, batch_ref, erep_ref,
                 bias_ref, lng_ref, lnb_ref, wg_ref, bg_ref, lngg_ref,
                 lngb_ref, wo_ref, bo_ref, lnlg_ref, lnlb_ref,
                 wn1_ref, bn1_ref, wn2_ref, bn2_ref, wn3_ref, bn3_ref,
                 wg1_ref, bg1_ref, wg2_ref, bg2_ref, wg3_ref, bg3_ref,
                 no_ref, pool_ref, go_ref):
    i = pl.program_id(0)
    h2 = _combine(parts_ref, table_ref, adst_ref, erep_ref, bias_ref,
                  lng_ref, lnb_ref)
    ge = _ln(jax.nn.relu(_dotd(h2, wg_ref[...]) + bg_ref[...]),
             lngg_ref[...], lngb_ref[...])
    bid = batch_ref[0, 0, :]
    mask = (jax.lax.broadcasted_iota(jnp.int32, (B, BN), 0)
            == bid[None, :]).astype(jnp.float32)
    geaug = jnp.concatenate(
        [ge, jnp.ones((BN, 1), jnp.float32),
         jnp.zeros((BN, 127), jnp.float32)], axis=1)
    part = _dot(mask, geaug)

    @pl.when(i == 0)
    def _():
        pool_ref[...] = jnp.zeros_like(pool_ref)

    pool_ref[...] += part

    ho = _ln(jax.nn.relu(_dotd(h2, wo_ref[...]) + bo_ref[...]),
             lnlg_ref[...], lnlb_ref[...])
    z = jax.nn.relu(_dotd(jnp.concatenate([x_ref[...], ho], axis=1),
                          wn1_ref[...]) + bn1_ref[...])
    z = jax.nn.relu(_dotd(z, wn2_ref[...]) + bn2_ref[...])
    no_ref[...] = _dotd(z, wn3_ref[...]) + bn3_ref[...]

    @pl.when(i == pl.num_programs(0) - 1)
    def _():
        pooled = pool_ref[...]
        gemean = pooled[:, 0:128] / jnp.maximum(pooled[:, 128:129], 1.0)
        zg = jax.nn.relu(_dotd(gemean, wg1_ref[...]) + bg1_ref[...])
        zg = jax.nn.relu(_dotd(zg, wg2_ref[...]) + bg2_ref[...])
        go_ref[...] = _dotd(zg, wg3_ref[...]) + bg3_ref[...]


def _blockdiag_att(att):
    maskm = jnp.kron(jnp.eye(H, dtype=jnp.float32),
                     jnp.ones((C, 1), jnp.float32))
    return att.reshape(H * C, 1) * maskm


def kernel(x, edge_attr, params, edge_index, batch):
    p = params
    src, dst = edge_index[0], edge_index[1]

    matts, a16s = [], []
    for l in range(2):
        g = p['gat'][l]
        matts.append(jnp.concatenate(
            [_blockdiag_att(g['att_src']), _blockdiag_att(g['att_dst'])],
            axis=1))
        a16s.append(g['att_e'].reshape(1, H * C))
    erep = jnp.kron(jnp.eye(H, dtype=jnp.float32),
                    jnp.ones((1, C), jnp.float32))
    kronm = jnp.kron(jnp.eye(H, dtype=jnp.float32),
                     jnp.ones((C, 1), jnp.float32))
    padc = lambda w, n: jnp.pad(w, ((0, 0), (0, n - w.shape[1])))
    padv = lambda v: jnp.pad(v, (0, 7)).reshape(1, 8)
    batch3d = batch.reshape(N // BN, 1, BN)

    grid_n = N // BN
    row128 = _row_spec(128)
    vec = lambda v, w: v.reshape(1, w)

    table1, adst1 = pl.pallas_call(
        _pre_kernel,
        grid=(grid_n,),
        in_specs=[row128, _full_spec((D, D)), _full_spec((1, D)),
                  _full_spec((2 * D, D)), _full_spec((D, 8))],
        out_specs=[_row_spec(NR), _row_spec(16)],
        out_shape=[jax.ShapeDtypeStruct((N, NR), jnp.float32),
                   jax.ShapeDtypeStruct((N, 16), jnp.float32)],
    )(x, p['W_in'], vec(p['b_in'], D), p['gat'][0]['W'], matts[0])

    e4pack = pl.pallas_call(
        _e4_kernel,
        grid=(E // BE,),
        in_specs=[pl.BlockSpec((BE, DE), lambda i: (i, 0)),
                  _full_spec((DE, D)), _full_spec((1, D)),
                  _full_spec((DE, D)), _full_spec((1, D)),
                  _full_spec((D, H))],
        out_specs=pl.BlockSpec((BE, 2 * H), lambda i: (i, 0)),
        out_shape=jax.ShapeDtypeStruct((E, 2 * H), jnp.float32),
    )(edge_attr, p['gat'][0]['W_e'], a16s[0], p['gat'][1]['W_e'], a16s[1],
      kronm)
    e4s = [e4pack[:, 0:4], e4pack[:, 4:8]]

    parts1 = _sc_edge_pass(table1, adst1, src, dst, e4s[0])
    g1 = p['gat'][0]
    table2, adst2 = pl.pallas_call(
        _mid_kernel,
        grid=(grid_n,),
        in_specs=[pl.BlockSpec((NC, BN, NR), lambda i: (0, i, 0)),
                  _row_spec(NR), _row_spec(16), row128,
                  _full_spec((H, D)), _full_spec((1, D)),
                  _full_spec((1, D)), _full_spec((1, D)),
                  _full_spec((2 * D, D)), _full_spec((D, 8))],
        out_specs=[_row_spec(NR), _row_spec(16)],
        out_shape=[jax.ShapeDtypeStruct((N, NR), jnp.float32),
                   jax.ShapeDtypeStruct((N, 16), jnp.float32)],
    )(parts1, table1, adst1, x, erep, vec(g1['bias'], D),
      vec(g1['ln_g'], D), vec(g1['ln_b'], D), p['gat'][1]['W'], matts[1])

    parts2 = _sc_edge_pass(table2, adst2, src, dst, e4s[1])
    g2 = p['gat'][1]
    no_full, pooled, go_full = pl.pallas_call(
        _post_kernel,
        grid=(grid_n,),
        in_specs=[pl.BlockSpec((NC, BN, NR), lambda i: (0, i, 0)),
                  _row_spec(NR), _row_spec(16), row128,
                  pl.BlockSpec((1, 1, BN), lambda i: (i, 0, 0)),
                  _full_spec((H, D)), _full_spec((1, D)),
                  _full_spec((1, D)), _full_spec((1, D)),
                  _full_spec((D, D)), _full_spec((1, D)),
                  _full_spec((1, D)), _full_spec((1, D)),
                  _full_spec((D, D)), _full_spec((1, D)),
                  _full_spec((1, D)), _full_spec((1, D)),
                  _full_spec((2 * D, 32)),
                  _full_spec((1, 32)), _full_spec((32, 8)),
                  _full_spec((1, 8)), _full_spec((8, 8)),
                  _full_spec((1, 8)),
                  _full_spec((D, 32)), _full_spec((1, 32)),
                  _full_spec((32, 8)), _full_spec((1, 8)),
                  _full_spec((8, 8)), _full_spec((1, 8))],
        out_specs=[pl.BlockSpec((BN, 8), lambda i: (i, 0)),
                   pl.BlockSpec((B, 256), lambda i: (0, 0)),
                   pl.BlockSpec((B, 8), lambda i: (0, 0))],
        out_shape=[jax.ShapeDtypeStruct((N, 8), jnp.float32),
                   jax.ShapeDtypeStruct((B, 256), jnp.float32),
                   jax.ShapeDtypeStruct((B, 8), jnp.float32)],
    )(parts2, table2, adst2, x, batch3d, erep, vec(g2['bias'], D),
      vec(g2['ln_g'], D), vec(g2['ln_b'], D),
      p['Wg'], vec(p['bg'], D), vec(p['lng_g'], D), vec(p['lng_b'], D),
      p['Wo'], vec(p['bo'], D), vec(p['lnl_g'], D), vec(p['lnl_b'], D),
      p['Wn1'], vec(p['bn1'], 32), p['Wn2'], vec(p['bn2'], 8),
      padc(p['Wn3'], 8), padv(p['bn3']),
      p['Wg1'], vec(p['bg1'], 32), p['Wg2'], vec(p['bg2'], 8),
      padc(p['Wg3'], 8), padv(p['bg3']))

    return jnp.concatenate([no_full[:, 0], go_full[:, 0]], axis=-1)


# speedup vs baseline: 1.1557x; 1.1408x over previous
import functools

import jax
import jax.numpy as jnp
from jax import lax
from jax.experimental import pallas as pl
from jax.experimental.pallas import tpu as pltpu
from jax.experimental.pallas import tpu_sc as plsc

N = 10000
E = 320000
D = 128
DE = 16
H = 4
C = 32
B = 64

NC = 2
NS = 16
NW = NC * NS
EW = E // NW
CH = 96
NCH = EW // CH
TAIL = EW - NCH * CH
NR = 144
ZTILE = 632
ZLAST = N - 15 * ZTILE


def _sc_edge_pass(table, adst, src, dst, e4):
    mesh = plsc.VectorSubcoreMesh(core_axis_name="c", subcore_axis_name="s")

    @functools.partial(
        pl.kernel,
        out_type=jax.ShapeDtypeStruct((NC, N, NR), jnp.float32),
        mesh=mesh,
        scratch_types=[
            pltpu.VMEM_SHARED((N, NR), jnp.float32),
            pltpu.VMEM((CH, NR), jnp.float32),
            pltpu.VMEM((CH, NR), jnp.float32),
            pltpu.VMEM((CH,), jnp.int32),
            pltpu.VMEM((CH,), jnp.int32),
            pltpu.VMEM((CH,), jnp.int32),
            pltpu.VMEM((CH,), jnp.int32),
            pltpu.VMEM((CH,), jnp.int32),
            pltpu.VMEM((CH,), jnp.int32),
            pltpu.VMEM((CH, 16), jnp.float32),
            pltpu.VMEM((CH, 16), jnp.float32),
            pltpu.VMEM((CH, 4), jnp.float32),
            pltpu.VMEM((CH, 4), jnp.float32),
            pltpu.VMEM((CH, 4), jnp.float32),
            pltpu.VMEM((CH, 4), jnp.float32),
            pltpu.VMEM((TAIL,), jnp.int32),
            pltpu.VMEM((TAIL,), jnp.int32),
            pltpu.SemaphoreType.DMA,
            pltpu.SemaphoreType.DMA,
            pltpu.SemaphoreType.DMA,
            pltpu.SemaphoreType.DMA,
            pltpu.SemaphoreType.DMA,
            pltpu.SemaphoreType.DMA,
        ],
        compiler_params=pltpu.CompilerParams(use_tc_tiling_on_sc=False,
                                             needs_layout_passes=False),
    )
    def body(table_h, adst_h, src_h, dst_h, e4_h, out_h,
             acc, rows0, rows1, src0, src1, dst0, dst1, dst2, dst3,
             adr0, adr1, e40, e41, e42, e43, srct, dstt,
             semr0, semr1, sema0, sema1, semi0, semi1):
        cid = lax.axis_index("c")
        sid = lax.axis_index("s")
        wid = sid * NC + cid
        lane = lax.broadcasted_iota(jnp.int32, (16,), 0)
        el = lane >> 2
        hh = lane & 3
        zv = jnp.zeros((16,), jnp.float32)
        rows = [rows0, rows1]
        srci = [src0, src1]
        dsti = [dst0, dst1, dst2, dst3]
        adr = [adr0, adr1]
        e4v = [e40, e41, e42, e43]
        semr = [semr0, semr1]
        sema = [sema0, sema1]
        semi = [semi0, semi1]

        def zrow(r, _):
            for cgrp in range(NR // 16):
                rows0[r, pl.ds(cgrp * 16, 16)] = zv
            return 0

        lax.fori_loop(0, CH, zrow, 0)
        tb = sid * ZTILE
        for k in range(5):
            off = pl.multiple_of(tb + k * CH, 8)
            pltpu.sync_copy(rows0, acc.at[pl.ds(off, CH)])
        off5 = pl.multiple_of(tb + 5 * CH, 8)
        off6 = pl.multiple_of(tb + 6 * CH, 8)

        @pl.when(sid < NS - 1)
        def _():
            pltpu.sync_copy(rows0, acc.at[pl.ds(off5, CH)])
            pltpu.sync_copy(rows0.at[pl.ds(0, ZTILE - 6 * CH)],
                            acc.at[pl.ds(off6, ZTILE - 6 * CH)])

        @pl.when(sid == NS - 1)
        def _():
            pltpu.sync_copy(rows0.at[pl.ds(0, ZLAST - 5 * CH)],
                            acc.at[pl.ds(off5, ZLAST - 5 * CH)])

        plsc.subcore_barrier()

        def load_idx_sync(g, sb, db):
            base = wid * EW + g * CH
            pltpu.sync_copy(src_h.at[pl.ds(base, CH)], srci[sb])
            pltpu.sync_copy(dst_h.at[pl.ds(base, CH)], dsti[db])
            pltpu.sync_copy(e4_h.at[pl.ds(base, CH)], e4v[db])

        def load_idx_async(g, sb, db):
            base = wid * EW + g * CH
            pltpu.async_copy(src_h.at[pl.ds(base, CH)], srci[sb], semi[sb])
            pltpu.async_copy(dst_h.at[pl.ds(base, CH)], dsti[db], semi[sb])
            pltpu.async_copy(e4_h.at[pl.ds(base, CH)], e4v[db], semi[sb])

        def wait_idx(g, sb, db):
            base = wid * EW + g * CH
            pltpu.make_async_copy(src_h.at[pl.ds(base, CH)], srci[sb],
                                  semi[sb]).wait()
            pltpu.make_async_copy(dst_h.at[pl.ds(base, CH)], dsti[db],
                                  semi[sb]).wait()
            pltpu.make_async_copy(e4_h.at[pl.ds(base, CH)], e4v[db],
                                  semi[sb]).wait()

        def start_gather(b, db):
            pltpu.async_copy(table_h.at[srci[b]], rows[b], semr[b])
            pltpu.async_copy(adst_h.at[dsti[db]], adr[b], sema[b])

        def wait_gather(b, db):
            pltpu.make_async_copy(table_h.at[srci[b]], rows[b], semr[b]).wait()
            pltpu.make_async_copy(adst_h.at[dsti[db]], adr[b], sema[b]).wait()

        def compute(nch4, rbuf, abuf, ebuf):
            @plsc.parallel_loop(0, nch4, unroll=2)
            def pstep(j):
                rowv = j * 4 + el
                asrc = plsc.load_gather(rbuf, [rowv, 128 + hh])
                ad = plsc.load_gather(abuf, [rowv, hh])
                ev = plsc.load_gather(ebuf, [rowv, hh])
                al = asrc + ad + ev
                al = jnp.maximum(al, al * 0.2)
                pv = jnp.exp(al)
                plsc.store_scatter(rbuf, [rowv, 128 + hh], pv)
                plsc.store_scatter(rbuf, [rowv, 132 + hh], ev)
                for t in range(4):
                    e = j * 4 + t
                    for h in range(H):
                        ps = pv[t * 4 + h]
                        for q in range(2):
                            off = h * 32 + q * 16
                            rbuf[e, pl.ds(off, 16)] = (
                                rbuf[e, pl.ds(off, 16)] * ps)

        load_idx_sync(0, 0, 0)
        start_gather(0, 0)
        load_idx_async(1, 1, 1)

        def quad(i, _):
            for gg in range(4):
                g = i * 4 + gg
                b = gg & 1
                nb = 1 - b
                db = gg
                dbn = (gg + 1) & 3
                db2 = (gg + 2) & 3

                wait_gather(b, db)

                @pl.when(g < NCH - 1)
                def _():
                    wait_idx(g + 1, nb, dbn)
                    start_gather(nb, dbn)

                @pl.when(g < NCH - 2)
                def _():
                    load_idx_async(g + 2, b, db2)

                compute(CH // 4, rows[b], adr[b], e4v[db])
                pltpu.sync_copy(rows[b], acc.at[dsti[db]], add=True)
            return 0

        lax.fori_loop(0, NCH // 4, quad, 0)

        tbase = wid * EW + NCH * CH
        pltpu.sync_copy(src_h.at[pl.ds(tbase, TAIL)], srct)
        pltpu.sync_copy(dst_h.at[pl.ds(tbase, TAIL)], dstt)
        pltpu.sync_copy(e4_h.at[pl.ds(tbase, TAIL)], e40.at[pl.ds(0, TAIL)])
        cp1 = pltpu.async_copy(table_h.at[srct], rows0.at[pl.ds(0, TAIL)],
                               semr0)
        cp2 = pltpu.async_copy(adst_h.at[dstt], adr0.at[pl.ds(0, TAIL)],
                               sema0)
        cp1.wait()
        cp2.wait()
        compute(TAIL // 4, rows0, adr0, e40)
        pltpu.sync_copy(rows0.at[pl.ds(0, TAIL)], acc.at[dstt], add=True)

        plsc.subcore_barrier()
        doff = pl.multiple_of(sid * ZTILE, 8)

        @pl.when(sid < NS - 1)
        def _():
            pltpu.sync_copy(acc.at[pl.ds(doff, ZTILE)],
                            out_h.at[cid, pl.ds(doff, ZTILE)])

        @pl.when(sid == NS - 1)
        def _():
            pltpu.sync_copy(acc.at[pl.ds(doff, ZLAST)],
                            out_h.at[cid, pl.ds(doff, ZLAST)])

    return body(table, adst, src, dst, e4)


BN = 400
BE = 4000
HIGH = jax.lax.Precision.HIGHEST


def _dot(a, b):
    return jnp.dot(a, b, precision=HIGH, preferred_element_type=jnp.float32)


def _dotd(a, b):
    return jnp.dot(a, b, preferred_element_type=jnp.float32)


def _ln(x, g, b, eps=1e-5):
    mu = jnp.mean(x, axis=-1, keepdims=True)
    var = jnp.mean((x - mu) ** 2, axis=-1, keepdims=True)
    return (x - mu) / jnp.sqrt(var + eps) * g + b


def _elu(x):
    return jnp.where(x > 0, x, jnp.exp(x) - 1.0)


def _row_spec(width):
    return pl.BlockSpec((BN, width), lambda i: (i, 0))


def _full_spec(shape):
    nd = len(shape)
    return pl.BlockSpec(shape, lambda i: (0,) * nd)


def _tables_from_xw(xw, matt):
    s = _dot(xw, matt)
    zero4 = jnp.zeros((xw.shape[0], 4), jnp.float32)
    one1 = jnp.ones((xw.shape[0], 1), jnp.float32)
    zero7 = jnp.zeros((xw.shape[0], 7), jnp.float32)
    zero12 = jnp.zeros((xw.shape[0], 12), jnp.float32)
    table = jnp.concatenate([xw, s[:, 0:4], zero4, one1, zero7], axis=1)
    adst16 = jnp.concatenate([s[:, 4:8], zero12], axis=1)
    return table, adst16


def _pre_kernel(x_ref, win_ref, bin_ref, w_ref, matt_ref,
                table_ref, adst_ref):
    x = x_ref[...]
    h0 = jax.nn.relu(_dotd(x, win_ref[...]) + bin_ref[...])
    xw = _dotd(jnp.concatenate([x, h0], axis=1), w_ref[...])
    table, adst16 = _tables_from_xw(xw, matt_ref[...])
    table_ref[...] = table
    adst_ref[...] = adst16


def _e4_kernel(ea_ref, we1_ref, att1_ref, we2_ref, att2_ref, kron_ref,
               o_ref):
    ea = ea_ref[...]
    e41 = _dot(_dotd(ea, we1_ref[...]) * att1_ref[...], kron_ref[...])
    e42 = _dot(_dotd(ea, we2_ref[...]) * att2_ref[...], kron_ref[...])
    o_ref[...] = jnp.concatenate([e41, e42], axis=1)


def _combine(parts_ref, table_ref, adst_ref, erep_ref, bias_ref, lng_ref,
             lnb_ref):
    s = parts_ref[0] + parts_ref[1]
    xw = table_ref[:, 0:128]
    a_src = table_ref[:, 128:132]
    a_dst = adst_ref[:, 0:4]
    num = s[:, 0:128]
    den = s[:, 128:132]
    se4 = s[:, 132:136]
    deg = s[:, 136:137]
    loop_a = se4 / jnp.maximum(deg, 1.0)
    al = a_src + a_dst + loop_a
    p_self = jnp.exp(jnp.maximum(al, 0.2 * al))
    den_t = _dot(den + p_self, erep_ref[...]) + 1e-16
    num_t = num + xw * _dot(p_self, erep_ref[...])
    out = num_t / den_t + bias_ref[...]
    return _elu(_ln(out, lng_ref[...], lnb_ref[...]))


def _mid_kernel(parts_ref, table_ref, adst_ref, x_ref, erep_ref, bias_ref,
                lng_ref, lnb_ref, w_ref, matt_ref,
                table2_ref, adst2_ref):
    h1 = _combine(parts_ref, table_ref, adst_ref, erep_ref, bias_ref,
                  lng_ref, lnb_ref)
    xw2 = _dotd(jnp.concatenate([x_ref[...], h1], axis=1), w_ref[...])
    table2, adst16 = _tables_from_xw(xw2, matt_ref[...])
    table2_ref[...] = table2
    adst2_ref[...] = adst16


def _post_kernel(parts_ref, table_ref, adst_ref, x_ref, batch_ref, erep_ref,
                 bias_ref, lng_ref, lnb_ref, wg_ref, bg_ref, lngg_ref,
                 lngb_ref, wo_ref, bo_ref, lnlg_ref, lnlb_ref,
                 wn1_ref, bn1_ref, wn2_ref, bn2_ref, wn3_ref, bn3_ref,
                 wg1_ref, bg1_ref, wg2_ref, bg2_ref, wg3_ref, bg3_ref,
                 no_ref, pool_ref, go_ref):
    i = pl.program_id(0)
    h2 = _combine(parts_ref, table_ref, adst_ref, erep_ref, bias_ref,
                  lng_ref, lnb_ref)
    ge = _ln(jax.nn.relu(_dotd(h2, wg_ref[...]) + bg_ref[...]),
             lngg_ref[...], lngb_ref[...])
    bid = batch_ref[0, 0, :]
    mask = (jax.lax.broadcasted_iota(jnp.int32, (B, BN), 0)
            == bid[None, :]).astype(jnp.float32)
    geaug = jnp.concatenate(
        [ge, jnp.ones((BN, 1), jnp.float32),
         jnp.zeros((BN, 127), jnp.float32)], axis=1)
    part = _dot(mask, geaug)

    @pl.when(i == 0)
    def _():
        pool_ref[...] = jnp.zeros_like(pool_ref)

    pool_ref[...] += part

    ho = _ln(jax.nn.relu(_dotd(h2, wo_ref[...]) + bo_ref[...]),
             lnlg_ref[...], lnlb_ref[...])
    z = jax.nn.relu(_dotd(jnp.concatenate([x_ref[...], ho], axis=1),
                          wn1_ref[...]) + bn1_ref[...])
    z = jax.nn.relu(_dotd(z, wn2_ref[...]) + bn2_ref[...])
    no_ref[...] = _dotd(z, wn3_ref[...]) + bn3_ref[...]

    @pl.when(i == pl.num_programs(0) - 1)
    def _():
        pooled = pool_ref[...]
        gemean = pooled[:, 0:128] / jnp.maximum(pooled[:, 128:129], 1.0)
        zg = jax.nn.relu(_dotd(gemean, wg1_ref[...]) + bg1_ref[...])
        zg = jax.nn.relu(_dotd(zg, wg2_ref[...]) + bg2_ref[...])
        go_ref[...] = _dotd(zg, wg3_ref[...]) + bg3_ref[...]


def _blockdiag_att(att):
    maskm = jnp.kron(jnp.eye(H, dtype=jnp.float32),
                     jnp.ones((C, 1), jnp.float32))
    return att.reshape(H * C, 1) * maskm


def kernel(x, edge_attr, params, edge_index, batch):
    p = params
    src, dst = edge_index[0], edge_index[1]

    matts, a16s = [], []
    for l in range(2):
        g = p['gat'][l]
        matts.append(jnp.concatenate(
            [_blockdiag_att(g['att_src']), _blockdiag_att(g['att_dst'])],
            axis=1))
        a16s.append(g['att_e'].reshape(1, H * C))
    erep = jnp.kron(jnp.eye(H, dtype=jnp.float32),
                    jnp.ones((1, C), jnp.float32))
    kronm = jnp.kron(jnp.eye(H, dtype=jnp.float32),
                     jnp.ones((C, 1), jnp.float32))
    padc = lambda w, n: jnp.pad(w, ((0, 0), (0, n - w.shape[1])))
    padv = lambda v: jnp.pad(v, (0, 7)).reshape(1, 8)
    batch3d = batch.reshape(N // BN, 1, BN)

    grid_n = N // BN
    row128 = _row_spec(128)
    vec = lambda v, w: v.reshape(1, w)

    table1, adst1 = pl.pallas_call(
        _pre_kernel,
        grid=(grid_n,),
        in_specs=[row128, _full_spec((D, D)), _full_spec((1, D)),
                  _full_spec((2 * D, D)), _full_spec((D, 8))],
        out_specs=[_row_spec(NR), _row_spec(16)],
        out_shape=[jax.ShapeDtypeStruct((N, NR), jnp.float32),
                   jax.ShapeDtypeStruct((N, 16), jnp.float32)],
    )(x, p['W_in'], vec(p['b_in'], D), p['gat'][0]['W'], matts[0])

    e4pack = pl.pallas_call(
        _e4_kernel,
        grid=(E // BE,),
        in_specs=[pl.BlockSpec((BE, DE), lambda i: (i, 0)),
                  _full_spec((DE, D)), _full_spec((1, D)),
                  _full_spec((DE, D)), _full_spec((1, D)),
                  _full_spec((D, H))],
        out_specs=pl.BlockSpec((BE, 2 * H), lambda i: (i, 0)),
        out_shape=jax.ShapeDtypeStruct((E, 2 * H), jnp.float32),
    )(edge_attr, p['gat'][0]['W_e'], a16s[0], p['gat'][1]['W_e'], a16s[1],
      kronm)
    e4s = [e4pack[:, 0:4], e4pack[:, 4:8]]

    parts1 = _sc_edge_pass(table1, adst1, src, dst, e4s[0])
    g1 = p['gat'][0]
    table2, adst2 = pl.pallas_call(
        _mid_kernel,
        grid=(grid_n,),
        in_specs=[pl.BlockSpec((NC, BN, NR), lambda i: (0, i, 0)),
                  _row_spec(NR), _row_spec(16), row128,
                  _full_spec((H, D)), _full_spec((1, D)),
                  _full_spec((1, D)), _full_spec((1, D)),
                  _full_spec((2 * D, D)), _full_spec((D, 8))],
        out_specs=[_row_spec(NR), _row_spec(16)],
        out_shape=[jax.ShapeDtypeStruct((N, NR), jnp.float32),
                   jax.ShapeDtypeStruct((N, 16), jnp.float32)],
    )(parts1, table1, adst1, x, erep, vec(g1['bias'], D),
      vec(g1['ln_g'], D), vec(g1['ln_b'], D), p['gat'][1]['W'], matts[1])

    parts2 = _sc_edge_pass(table2, adst2, src, dst, e4s[1])
    g2 = p['gat'][1]
    no_full, pooled, go_full = pl.pallas_call(
        _post_kernel,
        grid=(grid_n,),
        in_specs=[pl.BlockSpec((NC, BN, NR), lambda i: (0, i, 0)),
                  _row_spec(NR), _row_spec(16), row128,
                  pl.BlockSpec((1, 1, BN), lambda i: (i, 0, 0)),
                  _full_spec((H, D)), _full_spec((1, D)),
                  _full_spec((1, D)), _full_spec((1, D)),
                  _full_spec((D, D)), _full_spec((1, D)),
                  _full_spec((1, D)), _full_spec((1, D)),
                  _full_spec((D, D)), _full_spec((1, D)),
                  _full_spec((1, D)), _full_spec((1, D)),
                  _full_spec((2 * D, 32)),
                  _full_spec((1, 32)), _full_spec((32, 8)),
                  _full_spec((1, 8)), _full_spec((8, 8)),
                  _full_spec((1, 8)),
                  _full_spec((D, 32)), _full_spec((1, 32)),
                  _full_spec((32, 8)), _full_spec((1, 8)),
                  _full_spec((8, 8)), _full_spec((1, 8))],
        out_specs=[pl.BlockSpec((BN, 8), lambda i: (i, 0)),
                   pl.BlockSpec((B, 256), lambda i: (0, 0)),
                   pl.BlockSpec((B, 8), lambda i: (0, 0))],
        out_shape=[jax.ShapeDtypeStruct((N, 8), jnp.float32),
                   jax.ShapeDtypeStruct((B, 256), jnp.float32),
                   jax.ShapeDtypeStruct((B, 8), jnp.float32)],
    )(parts2, table2, adst2, x, batch3d, erep, vec(g2['bias'], D),
      vec(g2['ln_g'], D), vec(g2['ln_b'], D),
      p['Wg'], vec(p['bg'], D), vec(p['lng_g'], D), vec(p['lng_b'], D),
      p['Wo'], vec(p['bo'], D), vec(p['lnl_g'], D), vec(p['lnl_b'], D),
      p['Wn1'], vec(p['bn1'], 32), p['Wn2'], vec(p['bn2'], 8),
      padc(p['Wn3'], 8), padv(p['bn3']),
      p['Wg1'], vec(p['bg1'], 32), p['Wg2'], vec(p['bg2'], 8),
      padc(p['Wg3'], 8), padv(p['bg3']))

    return jnp.concatenate([no_full[:, 0], go_full[:, 0]], axis=-1)
